# baseline (speedup 1.0000x reference)
import functools

import jax
import jax.numpy as jnp
from jax import lax
from jax.experimental import pallas as pl
from jax.experimental.pallas import tpu as pltpu
from jax.experimental.pallas import tpu_sc as plsc

N = 10000
E = 160000
D_IN = 128
DM = 64
DH = 128
DE = 16
DXP = 16
DMV = 128
DK = 152
LAYERS = 4

_NC, _NS = 2, 16
_NW = _NC * _NS
_C = 128
_NCHUNK = E // _C
_ITERS = (_NCHUNK + _NW - 1) // _NW

_BE = 1000
_BN = 2000
_NPW = 312
_NLAST = N - _NPW * (_NW - 1)
_APW = _NLAST

f32 = jnp.float32
bf16 = jnp.bfloat16


def _sc_gather_body(h_h, xp_h, xn_h, src_h, dst_h, hh_h, xpg_h, xng_h,
                    sidx0, didx0, bufj0, bufi0, bufp0, bufn0, gsem0, wsem0,
                    sidx1, didx1, bufj1, bufi1, bufp1, bufn1, gsem1, wsem1):
    c = lax.axis_index("c")
    s = lax.axis_index("s")
    w = s * _NC + c
    S0 = (sidx0, didx0, bufj0, bufi0, bufp0, bufn0, gsem0, wsem0)
    S1 = (sidx1, didx1, bufj1, bufi1, bufp1, bufn1, gsem1, wsem1)

    def issue(i, St):
        sidx, didx, bufj, bufi, bufp, bufn, gsem, wsem = St
        j = i * _NW + w

        @pl.when(j < _NCHUNK)
        def _():
            base = pl.multiple_of(j * _C, _C)
            pltpu.sync_copy(src_h.at[pl.ds(base, _C)], sidx)
            pltpu.sync_copy(dst_h.at[pl.ds(base, _C)], didx)
            pltpu.async_copy(h_h.at[sidx], bufj, gsem)
            pltpu.async_copy(h_h.at[didx], bufi, gsem)
            pltpu.async_copy(xp_h.at[didx], bufp, gsem)
            pltpu.async_copy(xn_h.at[sidx], bufn, gsem)

    def drain(i, St):
        sidx, didx, bufj, bufi, bufp, bufn, gsem, wsem = St
        j = i * _NW + w

        @pl.when(j < _NCHUNK)
        def _():
            base = pl.multiple_of(j * _C, _C)
            pltpu.make_async_copy(h_h.at[sidx], bufj, gsem).wait()
            pltpu.make_async_copy(h_h.at[didx], bufi, gsem).wait()
            pltpu.make_async_copy(xp_h.at[didx], bufp, gsem).wait()
            pltpu.make_async_copy(xn_h.at[sidx], bufn, gsem).wait()
            pltpu.async_copy(bufj, hh_h.at[pl.ds(base, _C), pl.ds(0, DM)],
                             wsem)
            pltpu.async_copy(bufi, hh_h.at[pl.ds(base, _C), pl.ds(DM, DM)],
                             wsem)
            pltpu.async_copy(bufp, xpg_h.at[pl.ds(base, _C)], wsem)
            pltpu.async_copy(bufn, xng_h.at[pl.ds(base, _C)], wsem)

    def wait_wb(i, St):
        sidx, didx, bufj, bufi, bufp, bufn, gsem, wsem = St
        j = i * _NW + w

        @pl.when(j < _NCHUNK)
        def _():
            base = pl.multiple_of(j * _C, _C)
            pltpu.make_async_copy(
                bufj, hh_h.at[pl.ds(base, _C), pl.ds(0, DM)], wsem).wait()
            pltpu.make_async_copy(
                bufi, hh_h.at[pl.ds(base, _C), pl.ds(DM, DM)], wsem).wait()
            pltpu.make_async_copy(bufp, xpg_h.at[pl.ds(base, _C)],
                                  wsem).wait()
            pltpu.make_async_copy(bufn, xng_h.at[pl.ds(base, _C)],
                                  wsem).wait()

    K = _ITERS // 2
    issue(0, S0)
    issue(1, S1)
    drain(0, S0)

    def step(k, carry):
        i0 = 2 * k
        i1 = 2 * k + 1
        wait_wb(i0 - 2, S0)
        issue(i0, S0)
        drain(i0 - 1, S1)
        wait_wb(i1 - 2, S1)
        issue(i1, S1)
        drain(i1 - 1, S0)
        return carry

    lax.fori_loop(1, K, step, None)
    drain(_ITERS - 1, S1)
    wait_wb(_ITERS - 2, S0)
    wait_wb(_ITERS - 1, S1)


def _sc_scatter_body(mv_h, dsts_h, perm_h, wb_h, z_h, Mp_h,
                     wbuf, didx, pidx, buf, acc, sem):
    c = lax.axis_index("c")
    s = lax.axis_index("s")
    w = s * _NC + c

    nodelo = _NPW * w
    nodehi = jnp.where(w == _NW - 1, N, nodelo + _NPW)
    pltpu.sync_copy(wb_h, wbuf)
    pltpu.sync_copy(z_h, acc)
    wv = wbuf[pl.ds(8 * w, 16)]
    estart = wv[0]
    eend = wv[1]
    base0 = jnp.bitwise_and(estart, -8)
    nchunk = lax.div(eend - base0 + _C - 1, _C)

    def step(t, carry):
        base = pl.multiple_of(base0 + t * _C, 8)
        pltpu.sync_copy(dsts_h.at[pl.ds(base, _C)], didx)
        pltpu.sync_copy(perm_h.at[pl.ds(base, _C)], pidx)
        pltpu.async_copy(mv_h.at[pidx], buf, sem).wait()

        def grp(g, carry2):
            v = didx[pl.ds(16 * g, 16)]
            for i in range(16):
                n = v[i]
                loc = n - nodelo
                e = 16 * g + i

                @pl.when((n >= nodelo) & (n < nodehi))
                def _():
                    for k in range(DMV // 16):
                        sl = pl.ds(16 * k, 16)
                        acc[loc, sl] = acc[loc, sl] + buf[e, sl]
            return carry2

        lax.fori_loop(0, _C // 16, grp, None)
        return carry

    lax.fori_loop(0, nchunk, step, None)
    lo8 = pl.multiple_of(nodelo, 8)
    pltpu.sync_copy(acc.at[pl.ds(0, _NPW)], Mp_h.at[pl.ds(lo8, _NPW)])

    @pl.when(w == _NW - 1)
    def _():
        pltpu.sync_copy(acc.at[pl.ds(_NPW, _NLAST - _NPW)],
                        Mp_h.at[pl.ds(_NPW * _NW, _NLAST - _NPW)])


@functools.lru_cache(maxsize=1)
def _get_sc_kernels():
    mesh = plsc.VectorSubcoreMesh(core_axis_name="c", subcore_axis_name="s",
                                  num_cores=_NC, num_subcores=_NS)
    gbufs = [pltpu.VMEM((_C,), jnp.int32),
             pltpu.VMEM((_C,), jnp.int32),
             pltpu.VMEM((_C, DM), f32),
             pltpu.VMEM((_C, DM), f32),
             pltpu.VMEM((_C, DXP), f32),
             pltpu.VMEM((_C, DXP), f32),
             pltpu.SemaphoreType.DMA,
             pltpu.SemaphoreType.DMA]
    gather = pl.kernel(
        _sc_gather_body,
        out_type=[jax.ShapeDtypeStruct((E, 2 * DM), f32),
                  jax.ShapeDtypeStruct((E, DXP), f32),
                  jax.ShapeDtypeStruct((E, DXP), f32)],
        mesh=mesh,
        scratch_types=gbufs + gbufs,
        compiler_params=pltpu.CompilerParams(use_tc_tiling_on_sc=False),
    )
    scatter = pl.kernel(
        _sc_scatter_body,
        out_type=jax.ShapeDtypeStruct((N, DMV), f32),
        mesh=mesh,
        scratch_types=[pltpu.VMEM((8 * _NW + 16,), jnp.int32),
                       pltpu.VMEM((_C,), jnp.int32),
                       pltpu.VMEM((_C,), jnp.int32),
                       pltpu.VMEM((_C, DMV), f32),
                       pltpu.VMEM((_APW, DMV), f32),
                       pltpu.SemaphoreType.DMA],
    )
    return gather, scatter


def _dot(a, b):
    return jnp.dot(a.astype(bf16), b.astype(bf16),
                   preferred_element_type=f32)


def _prologue_body(nf, pW, pb, xp, h_o, xn_o):
    h_o[...] = jnp.maximum(_dot(nf[...], pW[...]) + pb[...], 0.0)
    xn_o[...] = -xp[...]


def _edge_body(hh, xpg, xng, ef, W1p, mask3, b1, mW2, mb2, xW1, xb1, xW2t,
               xbrow, mv_o):
    xdv = xpg[...] + xng[...]
    diff = xdv + 1e-6
    d2 = jnp.sum(diff * diff * mask3[...], axis=1, keepdims=True)
    nrow = hh.shape[0]
    z = jnp.zeros((nrow, DK - 2 * DM - DE - 1), f32)
    cat = jnp.concatenate([hh[...], ef[...], d2, z], axis=1)
    t = jnp.maximum(_dot(cat, W1p[...]) + b1[...], 0.0)
    m = _dot(t, mW2[...]) + mb2[...]
    t2 = jnp.maximum(_dot(m, xW1[...]) + xb1[...], 0.0)
    mx = _dot(t2, xW2t[...]) + xbrow[...]
    vec = -xdv * mx
    zv = jnp.zeros((nrow, DMV - DM - DXP), f32)
    mv_o[...] = jnp.concatenate([m, vec, zv], axis=1)


def _node_body(h, Mp, xp, nW1, nb1, nW2, nb2, hn_o, xpn_o, xnn_o):
    acc = Mp[...]
    M = acc[:, :DM]
    xdel = acc[:, DM:DM + DXP]
    hcat = jnp.concatenate([h[...], M], axis=1)
    z = jnp.maximum(_dot(hcat, nW1[...]) + nb1[...], 0.0)
    hn = _dot(z, nW2[...]) + nb2[...]
    xn = xp[...] + xdel
    hn_o[...] = hn
    xpn_o[...] = xn
    xnn_o[...] = -xn


def _readout_body(h, ro1, rb1, ro2, rb2, tW, tb, y_o):
    r = jnp.maximum(_dot(h[...], ro1[...]) + rb1[...], 0.0)
    r = _dot(r, ro2[...]) + rb2[...]
    srow = jnp.sum(r, axis=0, keepdims=True)
    y_o[...] = _dot(srow, tW[...]) + tb[...]


def _full(shape):
    return pl.BlockSpec(shape, lambda *_: tuple(0 for _ in shape))


_prologue = pl.pallas_call(
    _prologue_body,
    grid=(N // _BN,),
    in_specs=[pl.BlockSpec((_BN, D_IN), lambda i: (i, 0)),
              _full((D_IN, DM)), _full((1, DM)),
              pl.BlockSpec((_BN, DXP), lambda i: (i, 0))],
    out_specs=[pl.BlockSpec((_BN, DM), lambda i: (i, 0)),
               pl.BlockSpec((_BN, DXP), lambda i: (i, 0))],
    out_shape=[jax.ShapeDtypeStruct((N, DM), f32),
               jax.ShapeDtypeStruct((N, DXP), f32)],
)

_edge = pl.pallas_call(
    _edge_body,
    grid=(E // _BE,),
    in_specs=[pl.BlockSpec((_BE, 2 * DM), lambda i: (i, 0)),
              pl.BlockSpec((_BE, DXP), lambda i: (i, 0)),
              pl.BlockSpec((_BE, DXP), lambda i: (i, 0)),
              pl.BlockSpec((_BE, DE), lambda i: (i, 0)),
              _full((DK, DH)), _full((1, DXP)), _full((1, DH)),
              _full((DH, DM)), _full((1, DM)),
              _full((DM, DM)), _full((1, DM)),
              _full((DM, DXP)), _full((1, DXP))],
    out_specs=pl.BlockSpec((_BE, DMV), lambda i: (i, 0)),
    out_shape=jax.ShapeDtypeStruct((E, DMV), f32),
)

_node = pl.pallas_call(
    _node_body,
    grid=(N // _BN,),
    in_specs=[pl.BlockSpec((_BN, DM), lambda i: (i, 0)),
              pl.BlockSpec((_BN, DMV), lambda i: (i, 0)),
              pl.BlockSpec((_BN, DXP), lambda i: (i, 0)),
              _full((DH, DH)), _full((1, DH)),
              _full((DH, DM)), _full((1, DM))],
    out_specs=[pl.BlockSpec((_BN, DM), lambda i: (i, 0)),
               pl.BlockSpec((_BN, DXP), lambda i: (i, 0)),
               pl.BlockSpec((_BN, DXP), lambda i: (i, 0))],
    out_shape=[jax.ShapeDtypeStruct((N, DM), f32),
               jax.ShapeDtypeStruct((N, DXP), f32),
               jax.ShapeDtypeStruct((N, DXP), f32)],
)

_readout = pl.pallas_call(
    _readout_body,
    in_specs=[_full((N, DM)),
              _full((DM, DM)), _full((1, DM)),
              _full((DM, DM)), _full((1, DM)),
              _full((DM, DH)), _full((1, DH))],
    out_specs=_full((1, DH)),
    out_shape=jax.ShapeDtypeStruct((1, DH), f32),
)


def kernel(node_feats, edge_feats, x, params, edge_index):
    p = params
    src = edge_index[0]
    dst = edge_index[1]

    W1p = jnp.pad(p['m_W1'], ((0, DK - 2 * DM - DE - 1), (0, 0)))
    mask3 = (jnp.arange(DXP) < 3).astype(f32).reshape(1, DXP)
    xbrow = (jnp.zeros((DXP,), f32).at[:3].set(p['x_bias'])
             + p['x_b2']).reshape(1, DXP)
    xW2t = jnp.tile(p['x_W2'], (1, DXP))
    mb2 = (p['m_b2'] + p['m_bias']).reshape(1, DM)
    nb2 = (p['nm_b2'] + p['node_bias']).reshape(1, DM)
    xpad0 = jnp.pad(x, ((0, 0), (0, DXP - 3)))
    tWpad = jnp.pad(p['task_W'], ((0, 0), (0, DH - 1)))
    tbpad = jnp.pad(p['task_b'], (0, DH - 1)).reshape(1, DH)
    zacc = jnp.zeros((_APW, DMV), f32)

    perm = jnp.argsort(dst, stable=True).astype(jnp.int32)
    dsts = dst[perm]
    bounds = jnp.concatenate(
        [_NPW * jnp.arange(_NW, dtype=jnp.int32),
         jnp.array([N], jnp.int32)])
    wb0 = jnp.searchsorted(dsts, bounds).astype(jnp.int32)
    wb = jnp.pad(jnp.stack([wb0[:_NW], wb0[1:_NW + 1]], axis=1),
                 ((0, 0), (0, 6))).reshape(-1)
    wb = jnp.pad(wb, (0, 16))
    dsts_pad = jnp.concatenate([dsts, jnp.full((_C,), N, jnp.int32)])
    perm_pad = jnp.concatenate([perm, jnp.zeros((_C,), jnp.int32)])

    h, xneg = _prologue(node_feats, p['proj_W'],
                        p['proj_b'].reshape(1, DM), xpad0)
    xpad = xpad0
    sc_gather, sc_scatter = _get_sc_kernels()
    for _ in range(LAYERS):
        hh, xpg, xng = sc_gather(h, xpad, xneg, src, dst)
        mv = _edge(hh, xpg, xng, edge_feats, W1p, mask3,
                   p['m_b1'].reshape(1, DH), p['m_W2'], mb2,
                   p['x_W1'], p['x_b1'].reshape(1, DM), xW2t, xbrow)
        Mp = sc_scatter(mv, dsts_pad, perm_pad, wb, zacc)
        h, xpad, xneg = _node(h, Mp, xpad, p['nm_W1'],
                              p['nm_b1'].reshape(1, DH), p['nm_W2'], nb2)

    ypad = _readout(h, p['ro_W1'], p['ro_b1'].reshape(1, DM),
                    p['ro_W2'], p['ro_b2'].reshape(1, DM), tWpad, tbpad)
    return ypad[:, :1]
